# Initial kernel scaffold; baseline (speedup 1.0000x reference)
import jax
import jax.numpy as jnp
from jax.experimental import pallas as pl


def kernel(query, value, reference_points, vp_w, vp_b, so_w, so_b, aw_w, aw_b, spatial_shapes, level_start_index):
    raise NotImplementedError("write your pallas kernel here")


# speedup vs baseline: 147.6656x; 147.6656x over previous
import functools

import jax
import jax.numpy as jnp
from jax import lax
from jax.experimental import pallas as pl
from jax.experimental.pallas import tpu as pltpu
from jax.experimental.pallas import tpu_sc as plsc

H, W = 100, 176
NV = H * W
NH, NP, HD = 8, 8, 32
NQ = 40000
D = 256

_VROWS = 2200


def _vproj_body(v_ref, w_ref, b_ref, o_ref):
    v = jnp.dot(v_ref[...], w_ref[...], preferred_element_type=jnp.float32)
    v = v + b_ref[...]
    for h in range(NH):
        o_ref[h] = v[:, h * HD:(h + 1) * HD]


def _make_table(val, vp_wT, vp_b2):
    return pl.pallas_call(
        _vproj_body,
        grid=(NV // _VROWS,),
        in_specs=[
            pl.BlockSpec((_VROWS, D), lambda i: (i, 0)),
            pl.BlockSpec((D, D), lambda i: (0, 0)),
            pl.BlockSpec((1, D), lambda i: (0, 0)),
        ],
        out_specs=pl.BlockSpec((NH, _VROWS, HD), lambda i: (0, i, 0)),
        out_shape=jax.ShapeDtypeStruct((NH, NV, HD), jnp.float32),
    )(val, vp_wT, vp_b2)


_QROWS = 2000


def _prep_body(q_ref, r_ref, wc_ref, bc_ref, sel_ref,
               idxa_ref, idxb_ref, wgta_ref, wgtb_ref):
    qb = q_ref[...]
    hi = lax.Precision.HIGHEST
    big = jnp.dot(qb, wc_ref[...], precision=hi,
                  preferred_element_type=jnp.float32) + bc_ref[...]
    sox = big[:, 0:64]
    soy = big[:, 64:128]
    logits = big[:, 128:192]
    e = jnp.exp(logits)
    l0 = lax.broadcasted_iota(jnp.int32, (64, 64), 0)
    l1 = lax.broadcasted_iota(jnp.int32, (64, 64), 1)
    grp = (l0 // NP == l1 // NP).astype(jnp.float32)
    denom = jnp.dot(e, grp, precision=hi, preferred_element_type=jnp.float32)
    attn = e / denom
    rr = jnp.dot(r_ref[...], sel_ref[...], precision=hi,
                 preferred_element_type=jnp.float32)
    gx = rr[:, 0:64] + sox - 0.5
    gy = rr[:, 64:128] + soy - 0.5
    x0 = jnp.floor(gx)
    y0 = jnp.floor(gy)
    wx1 = gx - x0
    wy1 = gy - y0
    ax0 = (1.0 - wx1) * ((x0 >= 0.0) & (x0 <= W - 1.0)).astype(jnp.float32) * attn
    ax1 = wx1 * ((x0 >= -1.0) & (x0 <= W - 2.0)).astype(jnp.float32) * attn
    cy0 = (1.0 - wy1) * ((y0 >= 0.0) & (y0 <= H - 1.0)).astype(jnp.float32)
    cy1 = wy1 * ((y0 >= -1.0) & (y0 <= H - 2.0)).astype(jnp.float32)
    hl = (lax.broadcasted_iota(jnp.int32, (_QROWS, 64), 1) // NP) * NV
    x0i = jnp.clip(x0, 0.0, W - 1.0).astype(jnp.int32)
    y0i = jnp.clip(y0, 0.0, H - 1.0).astype(jnp.int32)
    x1i = jnp.clip(x0 + 1.0, 0.0, W - 1.0).astype(jnp.int32)
    y1i = jnp.clip(y0 + 1.0, 0.0, H - 1.0).astype(jnp.int32)
    r0 = y0i * W + hl
    r1 = y1i * W + hl
    idxa_ref[...] = jnp.concatenate([r0 + x0i, r0 + x1i], axis=1)
    idxb_ref[...] = jnp.concatenate([r1 + x0i, r1 + x1i], axis=1)
    wgta_ref[...] = jnp.concatenate([ax0 * cy0, ax1 * cy0], axis=1)
    wgtb_ref[...] = jnp.concatenate([ax0 * cy1, ax1 * cy1], axis=1)


def _make_idx_wgt(q, rxy, wcat, bcat, selxy):
    return pl.pallas_call(
        _prep_body,
        grid=(NQ // _QROWS,),
        in_specs=[
            pl.BlockSpec((_QROWS, D), lambda i: (i, 0)),
            pl.BlockSpec((_QROWS, 8), lambda i: (i, 0)),
            pl.BlockSpec((D, 192), lambda i: (0, 0)),
            pl.BlockSpec((1, 192), lambda i: (0, 0)),
            pl.BlockSpec((8, 128), lambda i: (0, 0)),
        ],
        out_specs=[
            pl.BlockSpec((_QROWS, 128), lambda i: (i, 0)),
            pl.BlockSpec((_QROWS, 128), lambda i: (i, 0)),
            pl.BlockSpec((_QROWS, 128), lambda i: (i, 0)),
            pl.BlockSpec((_QROWS, 128), lambda i: (i, 0)),
        ],
        out_shape=[
            jax.ShapeDtypeStruct((NQ, 128), jnp.int32),
            jax.ShapeDtypeStruct((NQ, 128), jnp.int32),
            jax.ShapeDtypeStruct((NQ, 128), jnp.float32),
            jax.ShapeDtypeStruct((NQ, 128), jnp.float32),
        ],
    )(q, rxy, wcat, bcat, selxy)


_CQ = 5
_GPC = _CQ * NH
_RPC = _GPC * 32
_SUB = 128


def _splat(i):
    return jnp.full((16,), i, jnp.int32)


def _sc_gather(table, idxa, idxb, wgta, wgtb):
    nc, ns = 2, 16
    nw = nc * ns
    qpw = NQ // nw
    nch = qpw // _CQ
    mesh = plsc.VectorSubcoreMesh(core_axis_name="c", subcore_axis_name="s",
                                  num_cores=nc, num_subcores=ns)

    @functools.partial(
        pl.kernel,
        out_type=jax.ShapeDtypeStruct((NQ * NH * HD,), jnp.float32),
        mesh=mesh,
        compiler_params=pltpu.CompilerParams(needs_layout_passes=False,
                                             use_tc_tiling_on_sc=False),
        scratch_types=[
            pltpu.VMEM((2 * _CQ * 256,), jnp.int32),
            pltpu.VMEM((2 * _CQ * 256,), jnp.float32),
            pltpu.VMEM((2 * _RPC,), jnp.int32),
            pltpu.VMEM((2 * _RPC, HD), jnp.float32),
            pltpu.VMEM((2 * _RPC,), jnp.float32),
            pltpu.SemaphoreType.DMA,
            pltpu.SemaphoreType.DMA,
            pltpu.SemaphoreType.DMA,
        ],
    )
    def k(table_hbm, idxa_hbm, idxb_hbm, wgta_hbm, wgtb_hbm, out_hbm,
          idx_in, wgt_in, sidx, rows, out_v, sem_in, sem_g, sem_out):
        wid = lax.axis_index("s") * nc + lax.axis_index("c")
        lanes = lax.iota(jnp.int32, 16)
        pat0 = (lanes // 8) * 64 + (lanes % 8)
        NIN = _CQ * 256
        HALF = _CQ * 128

        def in_refs(ci, buf):
            qlo = wid * qpw + ci * _CQ
            return [
                (idxa_hbm.at[pl.ds(qlo * 128, HALF)],
                 idx_in.at[pl.ds(buf * NIN, HALF)]),
                (idxb_hbm.at[pl.ds(qlo * 128, HALF)],
                 idx_in.at[pl.ds(buf * NIN + HALF, HALF)]),
                (wgta_hbm.at[pl.ds(qlo * 128, HALF)],
                 wgt_in.at[pl.ds(buf * NIN, HALF)]),
                (wgtb_hbm.at[pl.ds(qlo * 128, HALF)],
                 wgt_in.at[pl.ds(buf * NIN + HALF, HALF)]),
            ]

        def in_start(ci, buf):
            for s, d in in_refs(ci, buf):
                pltpu.async_copy(s, d, sem_in)

        def in_wait(ci, buf):
            for s, d in in_refs(ci, buf):
                pltpu.make_async_copy(s, d, sem_in).wait()

        def permute(buf):
            def perm_body(g, _):
                base = buf * NIN + (g // NH) * 128 + (g % NH) * 8
                o = buf * _RPC + g * 32
                sidx[pl.ds(o, 16)] = plsc.load_gather(idx_in, [base + pat0])
                sidx[pl.ds(o + 16, 16)] = plsc.load_gather(idx_in, [base + HALF + pat0])
                return 0
            lax.fori_loop(0, _GPC, perm_body, 0)

        def gather_refs(buf, j):
            o = buf * _RPC + j * _SUB
            return (table_hbm.at[sidx.at[pl.ds(o, _SUB)]],
                    rows.at[pl.ds(o, _SUB), :])

        def gather_start(buf):
            for j in range(_RPC // _SUB):
                s, d = gather_refs(buf, j)
                pltpu.async_copy(s, d, sem_g)

        def gather_wait(buf):
            for j in range(_RPC // _SUB):
                s, d = gather_refs(buf, j)
                pltpu.make_async_copy(s, d, sem_g).wait()

        def reduce(buf):
            def red_body(g, _):
                rbase = buf * _RPC + g * 32
                base = buf * NIN + (g // NH) * 128 + (g % NH) * 8
                w0 = plsc.load_gather(wgt_in, [base + pat0])
                w1 = plsc.load_gather(wgt_in, [base + HALF + pat0])
                zero = jnp.zeros((16,), jnp.float32)
                acc0 = [zero, zero, zero, zero]
                acc1 = [zero, zero, zero, zero]
                for s in range(32):
                    wsrc = w0 if s < 16 else w1
                    wv = wsrc.at[_splat(s % 16)].get(mode="promise_in_bounds")
                    k4 = s % 4
                    acc0[k4] = acc0[k4] + wv * rows[rbase + s, pl.ds(0, 16)]
                    acc1[k4] = acc1[k4] + wv * rows[rbase + s, pl.ds(16, 16)]
                out_v[pl.ds(rbase, 16)] = (acc0[0] + acc0[1]) + (acc0[2] + acc0[3])
                out_v[pl.ds(rbase + 16, 16)] = (acc1[0] + acc1[1]) + (acc1[2] + acc1[3])
                return 0
            lax.fori_loop(0, _GPC, red_body, 0)

        def out_refs(ci, buf):
            qlo = wid * qpw + ci * _CQ
            return (out_v.at[pl.ds(buf * _RPC, _RPC)],
                    out_hbm.at[pl.ds(qlo * 256, _RPC)])

        def out_start(ci, buf):
            s, d = out_refs(ci, buf)
            pltpu.async_copy(s, d, sem_out)

        def out_wait(ci, buf):
            s, d = out_refs(ci, buf)
            pltpu.make_async_copy(s, d, sem_out).wait()

        in_start(0, 0)
        in_wait(0, 0)
        permute(0)
        gather_start(0)
        in_start(1, 1)

        def body(i, _):
            cur = i % 2
            nxt = 1 - cur

            @pl.when(i + 1 < nch)
            def _():
                in_wait(i + 1, nxt)
                permute(nxt)

            gather_wait(cur)

            @pl.when(i + 1 < nch)
            def _():
                gather_start(nxt)

            reduce(cur)

            @pl.when(i > 0)
            def _():
                out_wait(i - 1, nxt)

            out_start(i, cur)

            @pl.when(i + 2 < nch)
            def _():
                in_start(i + 2, cur)
            return 0

        lax.fori_loop(0, nch, body, 0)
        out_wait(nch - 1, (nch - 1) % 2)

    return k(table, idxa, idxb, wgta, wgtb)


def kernel(query, value, reference_points, vp_w, vp_b, so_w, so_b, aw_w, aw_b,
           spatial_shapes, level_start_index):
    q = query[0]
    val = value[0]
    table = _make_table(val, vp_w.T, vp_b.reshape(1, D)).reshape(NH * NV, HD)
    rxy = jnp.concatenate(
        [reference_points[0, :, :, 0], reference_points[0, :, :, 1]], axis=1)
    wcat = jnp.concatenate([so_w[0::2].T, so_w[1::2].T, aw_w.T], axis=1)
    bcat = jnp.concatenate([so_b[0::2], so_b[1::2], aw_b]).reshape(1, 192)
    a = jnp.arange(4)
    l = jnp.arange(64)
    selq = (l[None, :] % 4 == a[:, None]).astype(jnp.float32)
    selxy = jnp.zeros((8, 128), jnp.float32)
    selxy = selxy.at[0:4, 0:64].set(selq * float(W))
    selxy = selxy.at[4:8, 64:128].set(selq * float(H))
    idxa, idxb, wgta, wgtb = _make_idx_wgt(q, rxy, wcat, bcat, selxy)
    out = _sc_gather(table, idxa.reshape(-1), idxb.reshape(-1),
                     wgta.reshape(-1), wgtb.reshape(-1))
    return out.reshape(1, NQ, D)
